# baseline (speedup 1.0000x reference)
import functools

import jax
import jax.numpy as jnp
from jax import lax
from jax.experimental import pallas as pl
from jax.experimental.pallas import tpu as pltpu
from jax.experimental.pallas import tpu_sc as plsc

KK = 8192
DD = 256
BETA_C = 0.25
N_ROWS = 4096
BM = 512


_CH = 512


def _argmin_body(f_ref, cb_ref, f2_ref, c2_ref, idx_ref, dsum_ref):
    i = pl.program_id(0)
    f2 = f2_ref[...]
    c2 = c2_ref[...]
    fb = (2.0 * f_ref[...]).astype(jnp.bfloat16)
    mm2 = lax.dot_general(fb, cb_ref[...].astype(jnp.bfloat16),
                          (((1,), (1,)), ((), ())),
                          preferred_element_type=jnp.float32)
    d = (f2 - mm2) + c2
    bm, kk = d.shape
    rv = d[:, 0:_CH]
    rc = jnp.zeros((bm, _CH), jnp.int32)
    for j in range(1, kk // _CH):
        dj = d[:, j * _CH:(j + 1) * _CH]
        lt = dj < rv
        rv = jnp.where(lt, dj, rv)
        rc = jnp.where(lt, j, rc)
    dmin = jnp.min(rv, axis=1, keepdims=True)
    gidx = rc * _CH + lax.broadcasted_iota(jnp.int32, (bm, _CH), 1)
    cand = jnp.where(rv == dmin, gidx, kk)
    idx_ref[...] = jnp.min(cand, axis=1).astype(jnp.int32)[:, None]
    block_sum = jnp.sum(dmin).reshape(1, 1)

    @pl.when(i == 0)
    def _():
        dsum_ref[...] = block_sum

    @pl.when(i != 0)
    def _():
        dsum_ref[...] = dsum_ref[...] + block_sum


def _argmin_call(flat, cb, f2, c2):
    return pl.pallas_call(
        _argmin_body,
        grid=(N_ROWS // BM,),
        in_specs=[
            pl.BlockSpec((BM, DD), lambda i: (i, 0)),
            pl.BlockSpec((KK, DD), lambda i: (0, 0)),
            pl.BlockSpec((BM, 1), lambda i: (i, 0)),
            pl.BlockSpec((1, KK), lambda i: (0, 0)),
        ],
        out_specs=[
            pl.BlockSpec((BM, 1), lambda i: (i, 0)),
            pl.BlockSpec((1, 1), lambda i: (0, 0)),
        ],
        out_shape=[
            jax.ShapeDtypeStruct((N_ROWS, 1), jnp.int32),
            jax.ShapeDtypeStruct((1, 1), jnp.float32),
        ],
    )(flat, cb, f2, c2)


@functools.lru_cache(maxsize=1)
def _make_sc_gather():
    try:
        info = plsc.get_sparse_core_info()
        nc, ns = info.num_cores, info.num_subcores
    except Exception:
        nc, ns = 2, 16
    nw = nc * ns
    b_per_w = N_ROWS // nw
    mesh = plsc.VectorSubcoreMesh(core_axis_name="c", subcore_axis_name="s")

    @functools.partial(
        pl.kernel,
        mesh=mesh,
        out_type=jax.ShapeDtypeStruct((N_ROWS, DD), jnp.float32),
        scratch_types=[
            pltpu.VMEM((b_per_w,), jnp.int32),
            pltpu.VMEM((b_per_w, DD), jnp.float32),
            pltpu.SemaphoreType.DMA,
        ],
    )
    def gather_k(table_hbm, idx_hbm, out_hbm, idx_v, rows_v, sem):
        wid = lax.axis_index("s") * nc + lax.axis_index("c")
        base = wid * b_per_w
        pltpu.sync_copy(idx_hbm.at[pl.ds(base, b_per_w)], idx_v)
        pltpu.async_copy(table_hbm.at[idx_v], rows_v, sem).wait()
        pltpu.sync_copy(rows_v, out_hbm.at[pl.ds(base, b_per_w)])

    return gather_k


def kernel(z_e, codebook):
    B, S, Dd = z_e.shape
    flat = z_e.reshape(-1, Dd)
    f2 = jnp.sum(flat ** 2, axis=1, keepdims=True)
    c2 = jnp.sum(codebook ** 2, axis=1, keepdims=True).T
    idx2d, dsum = _argmin_call(flat, codebook, f2, c2)
    indices = idx2d.reshape(-1)
    z_q = _make_sc_gather()(codebook, indices).reshape(B, S, Dd)
    m = dsum[0, 0] / jnp.float32(N_ROWS * DD)
    vq_loss = m + BETA_C * m
    return (z_q, indices.reshape(B, S), vq_loss)


# speedup vs baseline: 1.0755x; 1.0755x over previous
import functools

import jax
import jax.numpy as jnp
from jax import lax
from jax.experimental import pallas as pl
from jax.experimental.pallas import tpu as pltpu
from jax.experimental.pallas import tpu_sc as plsc

KK = 8192
DD = 256
BETA_C = 0.25
N_ROWS = 4096
BM = 1024


def _argmin_body(f_ref, cb_ref, f2_ref, c2_ref, idx_ref, dsum_ref):
    i = pl.program_id(0)
    f2 = f2_ref[...]
    c2 = c2_ref[...]
    fb = (2.0 * f_ref[...]).astype(jnp.bfloat16)
    mm2 = lax.dot_general(fb, cb_ref[...].astype(jnp.bfloat16),
                          (((1,), (1,)), ((), ())),
                          preferred_element_type=jnp.float32)
    d = (f2 - mm2) + c2
    dmin = jnp.min(d, axis=1, keepdims=True)
    iota = lax.broadcasted_iota(jnp.int32, d.shape, 1)
    cand = jnp.where(d == dmin, iota, d.shape[1])
    idx_ref[...] = jnp.min(cand, axis=1).astype(jnp.int32)[:, None]
    block_sum = jnp.sum(dmin).reshape(1, 1)

    @pl.when(i == 0)
    def _():
        dsum_ref[...] = block_sum

    @pl.when(i != 0)
    def _():
        dsum_ref[...] = dsum_ref[...] + block_sum


def _argmin_call(flat, cb, f2, c2):
    return pl.pallas_call(
        _argmin_body,
        grid=(N_ROWS // BM,),
        in_specs=[
            pl.BlockSpec((BM, DD), lambda i: (i, 0)),
            pl.BlockSpec((KK, DD), lambda i: (0, 0)),
            pl.BlockSpec((BM, 1), lambda i: (i, 0)),
            pl.BlockSpec((1, KK), lambda i: (0, 0)),
        ],
        out_specs=[
            pl.BlockSpec((BM, 1), lambda i: (i, 0)),
            pl.BlockSpec((1, 1), lambda i: (0, 0)),
        ],
        out_shape=[
            jax.ShapeDtypeStruct((N_ROWS, 1), jnp.int32),
            jax.ShapeDtypeStruct((1, 1), jnp.float32),
        ],
    )(flat, cb, f2, c2)


@functools.lru_cache(maxsize=1)
def _make_sc_gather():
    try:
        info = plsc.get_sparse_core_info()
        nc, ns = info.num_cores, info.num_subcores
    except Exception:
        nc, ns = 2, 16
    nw = nc * ns
    b_per_w = N_ROWS // nw
    mesh = plsc.VectorSubcoreMesh(core_axis_name="c", subcore_axis_name="s")

    @functools.partial(
        pl.kernel,
        mesh=mesh,
        out_type=jax.ShapeDtypeStruct((N_ROWS, DD), jnp.float32),
        scratch_types=[
            pltpu.VMEM((b_per_w,), jnp.int32),
            pltpu.VMEM((b_per_w, DD), jnp.float32),
            pltpu.SemaphoreType.DMA,
        ],
    )
    def gather_k(table_hbm, idx_hbm, out_hbm, idx_v, rows_v, sem):
        wid = lax.axis_index("s") * nc + lax.axis_index("c")
        base = wid * b_per_w
        pltpu.sync_copy(idx_hbm.at[pl.ds(base, b_per_w)], idx_v)
        pltpu.async_copy(table_hbm.at[idx_v], rows_v, sem).wait()
        pltpu.sync_copy(rows_v, out_hbm.at[pl.ds(base, b_per_w)])

    return gather_k


def kernel(z_e, codebook):
    B, S, Dd = z_e.shape
    flat = z_e.reshape(-1, Dd)
    f2 = jnp.sum(flat ** 2, axis=1, keepdims=True)
    c2 = jnp.sum(codebook ** 2, axis=1, keepdims=True).T
    idx2d, dsum = _argmin_call(flat, codebook, f2, c2)
    indices = idx2d.reshape(-1)
    z_q = _make_sc_gather()(codebook, indices).reshape(B, S, Dd)
    m = dsum[0, 0] / jnp.float32(N_ROWS * DD)
    vq_loss = m + BETA_C * m
    return (z_q, indices.reshape(B, S), vq_loss)


# speedup vs baseline: 1.1564x; 1.0752x over previous
import functools

import jax
import jax.numpy as jnp
from jax import lax
from jax.experimental import pallas as pl
from jax.experimental.pallas import tpu as pltpu
from jax.experimental.pallas import tpu_sc as plsc

KK = 8192
DD = 256
BETA_C = 0.25
N_ROWS = 4096
BM = 1024


def _argmin_body(f_ref, cb_ref, f2_ref, c2_ref, idx_ref, dsum_ref):
    i = pl.program_id(0)
    f2 = f2_ref[...]
    c2 = c2_ref[...]
    fb = (2.0 * f_ref[...]).astype(jnp.bfloat16)
    mm2 = lax.dot_general(fb, cb_ref[...].astype(jnp.bfloat16),
                          (((1,), (1,)), ((), ())),
                          preferred_element_type=jnp.float32)
    d = (f2 - mm2) + c2
    dmin = jnp.min(d, axis=1, keepdims=True)
    iota = lax.broadcasted_iota(jnp.int32, d.shape, 1).astype(jnp.float32)
    cand = jnp.where(d == dmin, iota, float(d.shape[1]))
    idx_ref[...] = jnp.min(cand, axis=1).astype(jnp.int32)[:, None]
    block_sum = jnp.sum(dmin).reshape(1, 1)

    @pl.when(i == 0)
    def _():
        dsum_ref[...] = block_sum

    @pl.when(i != 0)
    def _():
        dsum_ref[...] = dsum_ref[...] + block_sum

    @pl.when(i == (N_ROWS // BM) - 1)
    def _():
        m = dsum_ref[...] * (1.0 / float(N_ROWS * DD))
        dsum_ref[...] = m + BETA_C * m


def _argmin_call(flat, cb, f2, c2):
    return pl.pallas_call(
        _argmin_body,
        grid=(N_ROWS // BM,),
        in_specs=[
            pl.BlockSpec((BM, DD), lambda i: (i, 0)),
            pl.BlockSpec((KK, DD), lambda i: (0, 0)),
            pl.BlockSpec((BM, 1), lambda i: (i, 0)),
            pl.BlockSpec((1, KK), lambda i: (0, 0)),
        ],
        out_specs=[
            pl.BlockSpec((BM, 1), lambda i: (i, 0)),
            pl.BlockSpec((1, 1), lambda i: (0, 0)),
        ],
        out_shape=[
            jax.ShapeDtypeStruct((N_ROWS, 1), jnp.int32),
            jax.ShapeDtypeStruct((1, 1), jnp.float32),
        ],
    )(flat, cb, f2, c2)


@functools.lru_cache(maxsize=1)
def _make_sc_gather():
    try:
        info = plsc.get_sparse_core_info()
        nc, ns = info.num_cores, info.num_subcores
    except Exception:
        nc, ns = 2, 16
    nw = nc * ns
    b_per_w = N_ROWS // nw
    mesh = plsc.VectorSubcoreMesh(core_axis_name="c", subcore_axis_name="s")

    @functools.partial(
        pl.kernel,
        mesh=mesh,
        out_type=jax.ShapeDtypeStruct((N_ROWS, DD), jnp.float32),
        scratch_types=[
            pltpu.VMEM((b_per_w,), jnp.int32),
            pltpu.VMEM((b_per_w, DD), jnp.float32),
            pltpu.SemaphoreType.DMA,
        ],
    )
    def gather_k(table_hbm, idx_hbm, out_hbm, idx_v, rows_v, sem):
        wid = lax.axis_index("s") * nc + lax.axis_index("c")
        base = wid * b_per_w
        pltpu.sync_copy(idx_hbm.at[pl.ds(base, b_per_w)], idx_v)
        pltpu.async_copy(table_hbm.at[idx_v], rows_v, sem).wait()
        pltpu.sync_copy(rows_v, out_hbm.at[pl.ds(base, b_per_w)])

    return gather_k


def kernel(z_e, codebook):
    B, S, Dd = z_e.shape
    flat = z_e.reshape(-1, Dd)
    f2 = jnp.sum(flat ** 2, axis=1, keepdims=True)
    c2 = jnp.sum(codebook ** 2, axis=1, keepdims=True).T
    idx2d, vq2d = _argmin_call(flat, codebook, f2, c2)
    indices = idx2d.reshape(-1)
    z_q = _make_sc_gather()(codebook, indices).reshape(B, S, Dd)
    vq_loss = vq2d.reshape(())
    return (z_q, indices.reshape(B, S), vq_loss)


# speedup vs baseline: 1.1780x; 1.0187x over previous
import functools

import jax
import jax.numpy as jnp
from jax import lax
from jax.experimental import pallas as pl
from jax.experimental.pallas import tpu as pltpu
from jax.experimental.pallas import tpu_sc as plsc

KK = 8192
DD = 256
BETA_C = 0.25
N_ROWS = 4096
BM = 1024


def _argmin_body(f_ref, cb_ref, f2_ref, c2_ref, idx_ref, dsum_ref):
    i = pl.program_id(0)
    f2 = f2_ref[...]
    c2 = c2_ref[...]
    fb = (2.0 * f_ref[...]).astype(jnp.bfloat16)
    mm2 = lax.dot_general(fb, cb_ref[...].astype(jnp.bfloat16),
                          (((1,), (1,)), ((), ())),
                          preferred_element_type=jnp.float32)
    d = (f2 - mm2) + c2
    dmin = jnp.min(d, axis=1, keepdims=True)
    iota = lax.broadcasted_iota(jnp.int32, d.shape, 1).astype(jnp.float32)
    cand = jnp.where(d == dmin, iota, float(d.shape[1]))
    idx_ref[...] = jnp.min(cand, axis=1).astype(jnp.int32)
    block_sum = jnp.sum(dmin).reshape(1, 1)

    @pl.when(i == 0)
    def _():
        dsum_ref[...] = block_sum

    @pl.when(i != 0)
    def _():
        dsum_ref[...] = dsum_ref[...] + block_sum

    @pl.when(i == (N_ROWS // BM) - 1)
    def _():
        m = dsum_ref[...] * (1.0 / float(N_ROWS * DD))
        dsum_ref[...] = m + BETA_C * m


def _argmin_call(flat, cb, f2, c2):
    return pl.pallas_call(
        _argmin_body,
        grid=(N_ROWS // BM,),
        in_specs=[
            pl.BlockSpec((BM, DD), lambda i: (i, 0)),
            pl.BlockSpec((KK, DD), lambda i: (0, 0)),
            pl.BlockSpec((BM, 1), lambda i: (i, 0)),
            pl.BlockSpec((1, KK), lambda i: (0, 0)),
        ],
        out_specs=[
            pl.BlockSpec((BM,), lambda i: (i,)),
            pl.BlockSpec((1, 1), lambda i: (0, 0)),
        ],
        out_shape=[
            jax.ShapeDtypeStruct((N_ROWS,), jnp.int32),
            jax.ShapeDtypeStruct((1, 1), jnp.float32),
        ],
    )(flat, cb, f2, c2)


@functools.lru_cache(maxsize=1)
def _make_sc_gather():
    try:
        info = plsc.get_sparse_core_info()
        nc, ns = info.num_cores, info.num_subcores
    except Exception:
        nc, ns = 2, 16
    nw = nc * ns
    b_per_w = N_ROWS // nw
    mesh = plsc.VectorSubcoreMesh(core_axis_name="c", subcore_axis_name="s")

    @functools.partial(
        pl.kernel,
        mesh=mesh,
        out_type=jax.ShapeDtypeStruct((N_ROWS, DD), jnp.float32),
        scratch_types=[
            pltpu.VMEM((b_per_w,), jnp.int32),
            pltpu.VMEM((b_per_w, DD), jnp.float32),
            pltpu.SemaphoreType.DMA,
        ],
    )
    def gather_k(table_hbm, idx_hbm, out_hbm, idx_v, rows_v, sem):
        wid = lax.axis_index("s") * nc + lax.axis_index("c")
        base = wid * b_per_w
        pltpu.sync_copy(idx_hbm.at[pl.ds(base, b_per_w)], idx_v)
        pltpu.async_copy(table_hbm.at[idx_v], rows_v, sem).wait()
        pltpu.sync_copy(rows_v, out_hbm.at[pl.ds(base, b_per_w)])

    return gather_k


def kernel(z_e, codebook):
    B, S, Dd = z_e.shape
    flat = z_e.reshape(-1, Dd)
    f2 = jnp.sum(flat ** 2, axis=1, keepdims=True)
    c2 = jnp.sum(codebook ** 2, axis=1, keepdims=True).T
    indices, vq2d = _argmin_call(flat, codebook, f2, c2)
    z_q = _make_sc_gather()(codebook, indices).reshape(B, S, Dd)
    vq_loss = vq2d.reshape(())
    return (z_q, indices.reshape(B, S), vq_loss)
